# baseline (speedup 1.0000x reference)
import functools
import math

import jax
import jax.numpy as jnp
from jax import lax
from jax.experimental import pallas as pl
from jax.experimental.pallas import tpu as pltpu
from jax.experimental.pallas import tpu_sc as plsc

EMB = 32
SCALE = math.sqrt(EMB)

NC = 2
NS = 16
NW = NC * NS
LANES = 16

IDX_W = 128
CPB = 8
C = IDX_W * CPB


def _make_gather(B: int):
  assert B % (NW * C) == 0
  b_per_w = B // NW
  n_chunks = b_per_w // C
  chunk_rows_per_w = b_per_w // IDX_W

  mesh = plsc.VectorSubcoreMesh(core_axis_name="c", subcore_axis_name="s")

  @functools.partial(
      pl.kernel,
      out_type=jax.ShapeDtypeStruct((B, EMB), jnp.float32),
      mesh=mesh,
      scratch_types=[
          pltpu.VMEM((CPB, IDX_W), jnp.int32),
          pltpu.VMEM((C, EMB), jnp.float32),
          pltpu.SemaphoreType.DMA,
      ],
      compiler_params=pltpu.CompilerParams(use_tc_tiling_on_sc=False),
  )
  def body(idx_hbm, table_hbm, out_hbm, idx_v, rows_v, sem):
    wid = lax.axis_index("s") * NC + lax.axis_index("c")
    idx_row_base = wid * chunk_rows_per_w
    out_base = wid * b_per_w

    def chunk(g, _):
      pltpu.sync_copy(idx_hbm.at[pl.ds(idx_row_base + g * CPB, CPB)], idx_v)
      copies = []
      for j in range(CPB):
        copies.append(
            pltpu.async_copy(
                table_hbm.at[idx_v.at[j]],
                rows_v.at[pl.ds(j * IDX_W, IDX_W)],
                sem,
            ))
      for cp in copies:
        cp.wait()

      def scale(r, _):
        rows_v[r, pl.ds(0, LANES)] = rows_v[r, pl.ds(0, LANES)] * SCALE
        rows_v[r, pl.ds(LANES, LANES)] = rows_v[r, pl.ds(LANES, LANES)] * SCALE
        return ()

      lax.fori_loop(0, C, scale, ())
      pltpu.sync_copy(rows_v, out_hbm.at[pl.ds(out_base + g * C, C)])
      return ()

    lax.fori_loop(0, n_chunks, chunk, ())

  return body


def kernel(tokens, table):
  B = tokens.shape[0] * tokens.shape[1]
  idx2d = tokens.reshape(B // IDX_W, IDX_W)
  out = _make_gather(B)(idx2d, table)
  return out.reshape(tokens.shape[0], tokens.shape[1], EMB)


# speedup vs baseline: 1.2788x; 1.2788x over previous
import functools
import math

import jax
import jax.numpy as jnp
from jax import lax
from jax.experimental import pallas as pl
from jax.experimental.pallas import tpu as pltpu
from jax.experimental.pallas import tpu_sc as plsc

EMB = 32
SCALE = math.sqrt(EMB)

NC = 2
NS = 16
NW = NC * NS
LANES = 16

S = 50
BATCH = 16384
NB = BATCH // 128
BLK_PER_W = NB // NW


def _make_emb():
  mesh = plsc.VectorSubcoreMesh(core_axis_name="c", subcore_axis_name="s")

  @functools.partial(
      pl.kernel,
      out_type=jax.ShapeDtypeStruct((S, 4, NB, 8, 128), jnp.float32),
      mesh=mesh,
      scratch_types=[
          pltpu.VMEM((BLK_PER_W, 128), jnp.int32),
          pltpu.VMEM((128, EMB), jnp.float32),
          pltpu.VMEM((EMB, 128), jnp.float32),
          pltpu.SemaphoreType.DMA,
      ],
      compiler_params=pltpu.CompilerParams(
          use_tc_tiling_on_sc=False, needs_layout_passes=False),
  )
  def body(tok_hbm, table_hbm, out_hbm, idx_v, rows_v, t_v, sem):
    wid = lax.axis_index("s") * NC + lax.axis_index("c")
    blk0 = wid * BLK_PER_W
    iota = lax.iota(jnp.int32, LANES)
    iota_hi = iota + LANES

    def per_s(s, _):
      for j in range(BLK_PER_W):
        pltpu.sync_copy(tok_hbm.at[s, pl.ds((blk0 + j) * 128, 128)],
                        idx_v.at[j])
      for j in range(BLK_PER_W):
        pltpu.async_copy(table_hbm.at[idx_v.at[j]], rows_v, sem).wait()

        def per_tok(t, _):
          ln = jnp.full((LANES,), t, jnp.int32)
          plsc.store_scatter(t_v, [iota, ln],
                             rows_v[t, pl.ds(0, LANES)] * SCALE)
          plsc.store_scatter(t_v, [iota_hi, ln],
                             rows_v[t, pl.ds(LANES, LANES)] * SCALE)
          return ()

        lax.fori_loop(0, 128, per_tok, ())
        for te in range(4):
          pltpu.sync_copy(t_v.at[pl.ds(te * 8, 8)],
                          out_hbm.at[s, te, blk0 + j])
      return ()

    lax.fori_loop(0, S, per_s, ())

  return body


def kernel(tokens, table):
  out5 = _make_emb()(tokens.T, table)
  return out5.transpose(2, 4, 0, 1, 3).reshape(BATCH, S, EMB)


# speedup vs baseline: 1.4879x; 1.1636x over previous
import functools
import math

import jax
import jax.numpy as jnp
from jax import lax
from jax.experimental import pallas as pl
from jax.experimental.pallas import tpu as pltpu
from jax.experimental.pallas import tpu_sc as plsc

EMB = 32
SCALE = math.sqrt(EMB)

NC = 2
NS = 16
NW = NC * NS
LANES = 16

S = 50
BATCH = 16384
NB = BATCH // 128
BLK_PER_W = NB // NW
BPW = BLK_PER_W * 128


def _make_emb():
  mesh = plsc.VectorSubcoreMesh(core_axis_name="c", subcore_axis_name="s")

  @functools.partial(
      pl.kernel,
      out_type=jax.ShapeDtypeStruct((S, 4, NB, 8, 128), jnp.float32),
      mesh=mesh,
      scratch_types=[
          pltpu.VMEM((BLK_PER_W, 128), jnp.int32),
          pltpu.VMEM((128, EMB), jnp.float32),
          pltpu.VMEM((128, EMB), jnp.float32),
          pltpu.VMEM((EMB, 128), jnp.float32),
          pltpu.VMEM((EMB, 128), jnp.float32),
          pltpu.SemaphoreType.DMA,
          pltpu.SemaphoreType.DMA,
          pltpu.SemaphoreType.DMA,
          pltpu.SemaphoreType.DMA,
      ],
      compiler_params=pltpu.CompilerParams(
          use_tc_tiling_on_sc=False, needs_layout_passes=False),
  )
  def body(tok_hbm, table_hbm, out_hbm, idx_v, rows_a, rows_b,
           t_va, t_vb, gsem0, gsem1, osem0, osem1):
    wid = lax.axis_index("s") * NC + lax.axis_index("c")
    blk0 = wid * BLK_PER_W
    iota = lax.iota(jnp.int32, LANES)
    iota_hi = iota + LANES
    rows = (rows_a, rows_b)
    tvs = (t_va, t_vb)
    gsems = (gsem0, gsem1)
    osems = (osem0, osem1)

    def per_s(s, _):
      for j in range(BLK_PER_W):
        pltpu.sync_copy(tok_hbm.at[s, pl.ds((blk0 + j) * 128, 128)],
                        idx_v.at[j])

      def gather(j, slot):
        return pltpu.async_copy(table_hbm.at[idx_v.at[j]],
                                rows[slot], gsems[slot])

      g0 = gather(0, 0)
      g1 = gather(1, 1)
      out_copies = [None, None, None, None]
      for j in range(BLK_PER_W):
        p = j % 2
        rv = rows[p]
        tv = tvs[p]
        (g0 if p == 0 else g1).wait()
        if j >= 2:
          for cp in out_copies[j - 2]:
            cp.wait()

        def per_tok(i, _):
          for k in range(4):
            t = i * 4 + k
            ln = jnp.full((LANES,), t, jnp.int32)
            plsc.store_scatter(tv, [iota, ln],
                               rv[t, pl.ds(0, LANES)] * SCALE)
            plsc.store_scatter(tv, [iota_hi, ln],
                               rv[t, pl.ds(LANES, LANES)] * SCALE)
          return ()

        lax.fori_loop(0, 32, per_tok, ())

        if j + 2 < BLK_PER_W:
          if p == 0:
            g0 = gather(j + 2, 0)
          else:
            g1 = gather(j + 2, 1)
        out_copies[j] = [
            pltpu.async_copy(tv.at[pl.ds(te * 8, 8)],
                             out_hbm.at[s, te, blk0 + j], osems[p])
            for te in range(4)
        ]
      for j in (2, 3):
        for cp in out_copies[j]:
          cp.wait()
      return ()

    lax.fori_loop(0, S, per_s, ())

  return body


def kernel(tokens, table):
  out5 = _make_emb()(tokens.T, table)
  return out5.transpose(2, 4, 0, 1, 3).reshape(BATCH, S, EMB)


# speedup vs baseline: 1.5616x; 1.0495x over previous
import functools
import math

import jax
import jax.numpy as jnp
from jax import lax
from jax.experimental import pallas as pl
from jax.experimental.pallas import tpu as pltpu
from jax.experimental.pallas import tpu_sc as plsc

EMB = 32
VOCAB = 1000000
SCALE = math.sqrt(EMB)

NC = 2
NS = 16
NW = NC * NS
LANES = 16

S = 50
BATCH = 16384
NB = BATCH // 128
BLK_PER_W = NB // NW
BPW = BLK_PER_W * 128


def _make_emb():
  mesh = plsc.VectorSubcoreMesh(core_axis_name="c", subcore_axis_name="s")

  @functools.partial(
      pl.kernel,
      out_type=jax.ShapeDtypeStruct((S, 4, NB, 8, 128), jnp.float32),
      mesh=mesh,
      scratch_types=[
          pltpu.VMEM((BPW, S), jnp.int32),
          pltpu.VMEM((S * BLK_PER_W, 128), jnp.int32),
          pltpu.VMEM((BLK_PER_W, 128), jnp.int32),
          pltpu.VMEM((128, EMB), jnp.float32),
          pltpu.VMEM((128, EMB), jnp.float32),
          pltpu.VMEM((EMB, 128), jnp.float32),
          pltpu.VMEM((EMB, 128), jnp.float32),
          pltpu.SemaphoreType.DMA,
          pltpu.SemaphoreType.DMA,
          pltpu.SemaphoreType.DMA,
          pltpu.SemaphoreType.DMA,
      ],
      compiler_params=pltpu.CompilerParams(
          use_tc_tiling_on_sc=False, needs_layout_passes=False),
  )
  def body(tok_hbm, table_hbm, out_hbm, slab_v, slab_t, idx_v, rows_a,
           rows_b, t_va, t_vb, gsem0, gsem1, osem0, osem1):
    wid = lax.axis_index("s") * NC + lax.axis_index("c")
    blk0 = wid * BLK_PER_W
    iota = lax.iota(jnp.int32, LANES)
    iota_hi = iota + LANES
    rows = (rows_a, rows_b)
    tvs = (t_va, t_vb)
    gsems = (gsem0, gsem1)
    osems = (osem0, osem1)

    pltpu.sync_copy(tok_hbm.at[pl.ds(wid * BPW, BPW)], slab_v)
    iota2 = iota + iota
    iota4 = iota2 + iota2

    def tr(r, _):
      base = r // 128
      ll = jnp.full((LANES,), r % 128, jnp.int32)
      for off in (0, 16, 32, 34):
        jjoff = jnp.full((LANES,), base + off * BLK_PER_W, jnp.int32)
        plsc.store_scatter(slab_t, [iota4 + jjoff, ll],
                           slab_v[r, pl.ds(off, LANES)])
      return ()

    lax.fori_loop(0, BPW, tr, ())

    def per_s(s, _):
      def stage_idx(j, _):
        def cp16(g, _):
          v = slab_t[s * BLK_PER_W + j, pl.ds(g * LANES, LANES)]
          idx_v[j, pl.ds(g * LANES, LANES)] = jnp.minimum(
              jnp.maximum(v, 0), VOCAB - 1)
          return ()
        return lax.fori_loop(0, 128 // LANES, cp16, ())

      lax.fori_loop(0, BLK_PER_W, stage_idx, ())

      def gather(j, slot):
        return pltpu.async_copy(table_hbm.at[idx_v.at[j]],
                                rows[slot], gsems[slot])

      g0 = gather(0, 0)
      g1 = gather(1, 1)
      out_copies = [None, None, None, None]
      for j in range(BLK_PER_W):
        p = j % 2
        rv = rows[p]
        tv = tvs[p]
        (g0 if p == 0 else g1).wait()
        if j >= 2:
          for cp in out_copies[j - 2]:
            cp.wait()

        def per_tok(i, _):
          for k in range(4):
            t = i * 4 + k
            ln = jnp.full((LANES,), t, jnp.int32)
            plsc.store_scatter(tv, [iota, ln],
                               rv[t, pl.ds(0, LANES)] * SCALE)
            plsc.store_scatter(tv, [iota_hi, ln],
                               rv[t, pl.ds(LANES, LANES)] * SCALE)
          return ()

        lax.fori_loop(0, 32, per_tok, ())

        if j + 2 < BLK_PER_W:
          if p == 0:
            g0 = gather(j + 2, 0)
          else:
            g1 = gather(j + 2, 1)
        out_copies[j] = [
            pltpu.async_copy(tv.at[pl.ds(te * 8, 8)],
                             out_hbm.at[s, te, blk0 + j], osems[p])
            for te in range(4)
        ]
      for j in (2, 3):
        for cp in out_copies[j]:
          cp.wait()
      return ()

    lax.fori_loop(0, S, per_s, ())

  return body


def kernel(tokens, table):
  out5 = _make_emb()(tokens, table)
  return out5.transpose(2, 4, 0, 1, 3).reshape(BATCH, S, EMB)


# speedup vs baseline: 1.5948x; 1.0213x over previous
import functools
import math

import jax
import jax.numpy as jnp
from jax import lax
from jax.experimental import pallas as pl
from jax.experimental.pallas import tpu as pltpu
from jax.experimental.pallas import tpu_sc as plsc

EMB = 32
VOCAB = 1000000
SCALE = math.sqrt(EMB)

NC = 2
NS = 16
NW = NC * NS
LANES = 16

S = 50
BATCH = 16384
NB = BATCH // 128
BLK_PER_W = NB // NW
BPW = BLK_PER_W * 128


def _make_emb():
  mesh = plsc.VectorSubcoreMesh(core_axis_name="c", subcore_axis_name="s")

  @functools.partial(
      pl.kernel,
      out_type=jax.ShapeDtypeStruct((S, 4, NB, 8, 128), jnp.float32),
      mesh=mesh,
      scratch_types=[
          pltpu.VMEM((7, BLK_PER_W, 8, 128), jnp.int32),
          pltpu.VMEM((BLK_PER_W, 128), jnp.int32),
          pltpu.VMEM((128, EMB), jnp.float32),
          pltpu.VMEM((128, EMB), jnp.float32),
          pltpu.VMEM((EMB, 128), jnp.float32),
          pltpu.VMEM((EMB, 128), jnp.float32),
          pltpu.SemaphoreType.DMA,
          pltpu.SemaphoreType.DMA,
          pltpu.SemaphoreType.DMA,
          pltpu.SemaphoreType.DMA,
      ],
      compiler_params=pltpu.CompilerParams(
          use_tc_tiling_on_sc=False, needs_layout_passes=False),
  )
  def body(tok_hbm, table_hbm, out_hbm, slab4, idx_v, rows_a,
           rows_b, t_va, t_vb, gsem0, gsem1, osem0, osem1):
    wid = lax.axis_index("s") * NC + lax.axis_index("c")
    blk0 = wid * BLK_PER_W
    iota = lax.iota(jnp.int32, LANES)
    iota_hi = iota + LANES
    rows = (rows_a, rows_b)
    tvs = (t_va, t_vb)
    gsems = (gsem0, gsem1)
    osems = (osem0, osem1)

    pltpu.sync_copy(tok_hbm.at[:, pl.ds(blk0, BLK_PER_W)], slab4)

    def per_s(s, _):
      sr = s // 8
      sl = s % 8

      def stage_idx(j, _):
        def cp16(g, _):
          v = slab4[sr, j, sl, pl.ds(g * LANES, LANES)]
          idx_v[j, pl.ds(g * LANES, LANES)] = jnp.minimum(
              jnp.maximum(v, 0), VOCAB - 1)
          return ()
        return lax.fori_loop(0, 128 // LANES, cp16, ())

      lax.fori_loop(0, BLK_PER_W, stage_idx, ())

      def gather(j, slot):
        return pltpu.async_copy(table_hbm.at[idx_v.at[j]],
                                rows[slot], gsems[slot])

      g0 = gather(0, 0)
      g1 = gather(1, 1)
      out_copies = [None, None, None, None]
      for j in range(BLK_PER_W):
        p = j % 2
        rv = rows[p]
        tv = tvs[p]
        (g0 if p == 0 else g1).wait()
        if j >= 2:
          for cp in out_copies[j - 2]:
            cp.wait()

        def per_tok(i, _):
          for k in range(4):
            t = i * 4 + k
            ln = jnp.full((LANES,), t, jnp.int32)
            plsc.store_scatter(tv, [iota, ln],
                               rv[t, pl.ds(0, LANES)] * SCALE)
            plsc.store_scatter(tv, [iota_hi, ln],
                               rv[t, pl.ds(LANES, LANES)] * SCALE)
          return ()

        lax.fori_loop(0, 32, per_tok, ())

        if j + 2 < BLK_PER_W:
          if p == 0:
            g0 = gather(j + 2, 0)
          else:
            g1 = gather(j + 2, 1)
        out_copies[j] = [
            pltpu.async_copy(tv.at[pl.ds(te * 8, 8)],
                             out_hbm.at[s, te, blk0 + j], osems[p])
            for te in range(4)
        ]
      for j in (2, 3):
        for cp in out_copies[j]:
          cp.wait()
      return ()

    lax.fori_loop(0, S, per_s, ())

  return body


def kernel(tokens, table):
  tp = jnp.pad(tokens, ((0, 0), (0, 6)))
  t4 = tp.T.reshape(7, 8, NB, 128).transpose(0, 2, 1, 3)
  out5 = _make_emb()(t4, table)
  return out5.transpose(2, 4, 0, 1, 3).reshape(BATCH, S, EMB)


# speedup vs baseline: 1.8230x; 1.1431x over previous
import functools
import math

import jax
import jax.numpy as jnp
from jax import lax
from jax.experimental import pallas as pl
from jax.experimental.pallas import tpu as pltpu
from jax.experimental.pallas import tpu_sc as plsc

EMB = 32
VOCAB = 1000000
SCALE = math.sqrt(EMB)

NC = 2
NS = 16
NW = NC * NS
LANES = 16

S = 50
BATCH = 16384
NB = BATCH // 128
BLK_PER_W = NB // NW
BPW = BLK_PER_W * 128


def _make_emb():
  mesh = plsc.VectorSubcoreMesh(core_axis_name="c", subcore_axis_name="s")

  @functools.partial(
      pl.kernel,
      out_type=jax.ShapeDtypeStruct((S, 4, NB, 8, 128), jnp.float32),
      mesh=mesh,
      scratch_types=[
          pltpu.VMEM((7, BLK_PER_W, 8, 128), jnp.int32),
          pltpu.VMEM((BLK_PER_W, 128), jnp.int32),
          pltpu.VMEM((128, EMB), jnp.float32),
          pltpu.VMEM((128, EMB), jnp.float32),
          pltpu.VMEM((EMB, 128), jnp.float32),
          pltpu.VMEM((EMB, 128), jnp.float32),
          pltpu.VMEM((EMB, 256), jnp.float32),
          pltpu.SemaphoreType.DMA,
          pltpu.SemaphoreType.DMA,
          pltpu.SemaphoreType.DMA,
          pltpu.SemaphoreType.DMA,
      ],
      compiler_params=pltpu.CompilerParams(
          use_tc_tiling_on_sc=False, needs_layout_passes=False),
  )
  def body(tok_hbm, table_hbm, out_hbm, slab4, idx_v, rows_a,
           rows_b, t_va, t_vb, tsw, gsem0, gsem1, osem0, osem1):
    wid = lax.axis_index("s") * NC + lax.axis_index("c")
    blk0 = wid * BLK_PER_W
    iota = lax.iota(jnp.int32, LANES)
    iota_hi = iota + LANES
    rows = (rows_a, rows_b)
    tvs = (t_va, t_vb)
    gsems = (gsem0, gsem1)
    osems = (osem0, osem1)

    pltpu.sync_copy(tok_hbm.at[:, pl.ds(blk0, BLK_PER_W)], slab4)

    def per_s(s, _):
      sr = s // 8
      sl = s % 8

      def stage_idx(j, _):
        def cp16(g, _):
          v = slab4[sr, j, sl, pl.ds(g * LANES, LANES)]
          idx_v[j, pl.ds(g * LANES, LANES)] = jnp.minimum(
              jnp.maximum(v, 0), VOCAB - 1)
          return ()
        return lax.fori_loop(0, 128 // LANES, cp16, ())

      lax.fori_loop(0, BLK_PER_W, stage_idx, ())

      def gather(j, slot):
        return pltpu.async_copy(table_hbm.at[idx_v.at[j]],
                                rows[slot], gsems[slot])

      g0 = gather(0, 0)
      g1 = gather(1, 1)
      out_copies = [None, None, None, None]
      for j in range(BLK_PER_W):
        p = j % 2
        rv = rows[p]
        tv = tvs[p]
        (g0 if p == 0 else g1).wait()
        if j >= 2:
          for cp in out_copies[j - 2]:
            cp.wait()

        def per_tok(i, _):
          for k in range(4):
            t = i * 4 + k
            ln = jnp.full((LANES,), t, jnp.int32)
            plsc.store_scatter(tsw, [iota, ln + iota],
                               rv[t, pl.ds(0, LANES)] * SCALE)
            plsc.store_scatter(tsw, [iota_hi, ln + iota_hi],
                               rv[t, pl.ds(LANES, LANES)] * SCALE)
          return ()

        lax.fori_loop(0, 32, per_tok, ())

        def unskew(e, _):
          for g in range(128 // LANES):
            tv[e, pl.ds(g * LANES, LANES)] = (
                tsw[e, pl.ds(e + g * LANES, LANES)])
          return ()

        lax.fori_loop(0, EMB, unskew, ())

        if j + 2 < BLK_PER_W:
          if p == 0:
            g0 = gather(j + 2, 0)
          else:
            g1 = gather(j + 2, 1)
        out_copies[j] = [
            pltpu.async_copy(tv.at[pl.ds(te * 8, 8)],
                             out_hbm.at[s, te, blk0 + j], osems[p])
            for te in range(4)
        ]
      for j in (2, 3):
        for cp in out_copies[j]:
          cp.wait()
      return ()

    lax.fori_loop(0, S, per_s, ())

  return body


def kernel(tokens, table):
  tp = jnp.pad(tokens, ((0, 0), (0, 6)))
  t4 = tp.T.reshape(7, 8, NB, 128).transpose(0, 2, 1, 3)
  out5 = _make_emb()(t4, table)
  return out5.transpose(2, 4, 0, 1, 3).reshape(BATCH, S, EMB)
